# Initial kernel scaffold; baseline (speedup 1.0000x reference)
import jax
import jax.numpy as jnp
from jax.experimental import pallas as pl


def kernel(f2d, msa, msa_emb, gamma, beta, w_conv, b_conv, table, w_le, b_le):
    raise NotImplementedError("write your pallas kernel here")


# speedup vs baseline: 1.0107x; 1.0107x over previous
import functools

import jax
import jax.numpy as jnp
from jax import lax
from jax.experimental import pallas as pl
from jax.experimental.pallas import tpu as pltpu


def _norm_elu_conv_kernel(x_ref, g_ref, be_ref, w_ref, bc_ref, o_ref,
                          *, cin, inv_p):
    x = x_ref[...]
    lw = x.shape[1]
    s1 = jnp.sum(x, axis=0, keepdims=True)
    s2 = jnp.sum(x * x, axis=0, keepdims=True)
    ka = lax.broadcasted_iota(jnp.int32, (lw, lw), 0) % cin
    la = lax.broadcasted_iota(jnp.int32, (lw, lw), 1) % cin
    fold = (ka == la).astype(jnp.float32)
    mean = jnp.dot(s1, fold, preferred_element_type=jnp.float32) * inv_p
    ex2 = jnp.dot(s2, fold, preferred_element_type=jnp.float32) * inv_p
    var = jnp.maximum(ex2 - mean * mean, 0.0)
    rstd = lax.rsqrt(var + 1e-5)
    scale = g_ref[...] * rstd
    shift = be_ref[...] - mean * scale
    xa = x * scale + shift
    xe = jnp.where(xa > 0, xa, jnp.exp(jnp.minimum(xa, 0.0)) - 1.0)
    y = jnp.dot(xe, w_ref[...], preferred_element_type=jnp.float32)
    o_ref[...] = y + bc_ref[...]


def _msa_embed_kernel(tok_ref, emb_ref, tab_ref, w_ref, o_ref, *, d, ge, vocab):
    rows = tok_ref.shape[0]
    dl = ge * d
    sub = lax.broadcasted_iota(jnp.int32, (rows, dl), 1) // d
    tok = tok_ref[...]
    tid = jnp.zeros((rows, dl), jnp.int32)
    for j in range(ge):
        tid = tid + jnp.where(sub == j, tok[:, j:j + 1], 0)
    acc = jnp.dot(emb_ref[...], w_ref[...], preferred_element_type=jnp.float32)
    for t in range(vocab):
        acc = acc + jnp.where(tid == t, tab_ref[t:t + 1, :], 0.0)
    o_ref[...] = acc


def kernel(f2d, msa, msa_emb, gamma, beta, w_conv, b_conv, table, w_le, b_le):
    B, L, _, cin = f2d.shape
    d = w_conv.shape[0]
    p = L * L
    g = 128 // cin
    lw = g * cin
    dout = g * d
    pd = p // g

    x_f = f2d.astype(jnp.float32).reshape(B, pd, lw)
    g_f = jnp.tile(gamma.astype(jnp.float32), g).reshape(1, lw)
    be_f = jnp.tile(beta.astype(jnp.float32), g).reshape(1, lw)
    w_big = jnp.kron(jnp.eye(g, dtype=jnp.float32),
                     jnp.transpose(w_conv).astype(jnp.float32))
    bc_big = jnp.tile(b_conv.astype(jnp.float32), g).reshape(1, dout)

    body = functools.partial(_norm_elu_conv_kernel, cin=cin, inv_p=1.0 / p)
    x_out = pl.pallas_call(
        body,
        out_shape=jax.ShapeDtypeStruct((B, pd, dout), jnp.float32),
        grid=(B,),
        in_specs=[
            pl.BlockSpec((None, pd, lw), lambda b: (b, 0, 0)),
            pl.BlockSpec((1, lw), lambda b: (0, 0)),
            pl.BlockSpec((1, lw), lambda b: (0, 0)),
            pl.BlockSpec((lw, dout), lambda b: (0, 0)),
            pl.BlockSpec((1, dout), lambda b: (0, 0)),
        ],
        out_specs=pl.BlockSpec((None, pd, dout), lambda b: (b, 0, 0)),
        compiler_params=pltpu.CompilerParams(
            dimension_semantics=("parallel",),
            vmem_limit_bytes=64 * 1024 * 1024,
        ),
    )(x_f, g_f, be_f, w_big, bc_big)
    x = x_out.reshape(B, L, L, d)

    Bm, Nm, Lm = msa.shape
    r = Bm * Nm * Lm
    demb = msa_emb.shape[-1]
    vocab = table.shape[0]
    ge = 128 // d
    le = ge * demb
    dl = ge * d
    rf = r // ge

    tok_f = msa.reshape(rf, ge).astype(jnp.int32)
    emb_f = msa_emb.astype(jnp.float32).reshape(rf, le)
    tab = (jnp.tile(table.astype(jnp.float32), (1, ge))
           + jnp.tile(b_le.astype(jnp.float32), ge)[None, :])
    wle_big = jnp.kron(jnp.eye(ge, dtype=jnp.float32),
                       jnp.transpose(w_le).astype(jnp.float32))

    tile = min(1024, rf)
    body2 = functools.partial(_msa_embed_kernel, d=d, ge=ge, vocab=vocab)
    m_out = pl.pallas_call(
        body2,
        out_shape=jax.ShapeDtypeStruct((rf, dl), jnp.float32),
        grid=(pl.cdiv(rf, tile),),
        in_specs=[
            pl.BlockSpec((tile, ge), lambda i: (i, 0)),
            pl.BlockSpec((tile, le), lambda i: (i, 0)),
            pl.BlockSpec((vocab, dl), lambda i: (0, 0)),
            pl.BlockSpec((le, dl), lambda i: (0, 0)),
        ],
        out_specs=pl.BlockSpec((tile, dl), lambda i: (i, 0)),
        compiler_params=pltpu.CompilerParams(
            dimension_semantics=("parallel",),
            vmem_limit_bytes=64 * 1024 * 1024,
        ),
    )(tok_f, emb_f, tab, wle_big)
    m = m_out.reshape(Bm, Nm, Lm, d)

    return x, m


# speedup vs baseline: 2.7631x; 2.7337x over previous
import jax
import jax.numpy as jnp
from jax import lax
from jax.experimental import pallas as pl
from jax.experimental.pallas import tpu as pltpu


def _norm_elu_conv_kernel(x_ref, sc_ref, sh_ref, w_ref, b_ref, o_ref):
    rs, cin, L = x_ref.shape
    x = x_ref[...].reshape(rs * cin, L)
    xa = x * sc_ref[...] + sh_ref[...]
    xe = jnp.where(xa > 0, xa, jnp.exp(jnp.minimum(xa, 0.0)) - 1.0)
    y = jnp.dot(w_ref[...], xe, preferred_element_type=jnp.float32) + b_ref[...]
    o_ref[...] = y.reshape(o_ref.shape)


def _msa_embed_kernel(tok_ref, emb_ref, w_ref, c_ref, b_ref, o_ref, *, vocab):
    ns, demb, L = emb_ref.shape
    e = emb_ref[...].reshape(ns * demb, L)
    y = jnp.dot(w_ref[...], e, preferred_element_type=jnp.float32) + b_ref[...]
    tok = tok_ref[...]
    masks = jnp.concatenate(
        [(tok == t).astype(jnp.float32) for t in range(vocab)], axis=0)
    y = y + jnp.dot(c_ref[...], masks, preferred_element_type=jnp.float32)
    o_ref[...] = y.reshape(o_ref.shape)


def kernel(f2d, msa, msa_emb, gamma, beta, w_conv, b_conv, table, w_le, b_le):
    f32 = jnp.float32
    B, L, _, cin = f2d.shape
    d = w_conv.shape[0]

    x_t = jnp.transpose(f2d.astype(f32), (0, 1, 3, 2))

    mean = jnp.mean(x_t, axis=(1, 3))
    ex2 = jnp.mean(x_t * x_t, axis=(1, 3))
    var = jnp.maximum(ex2 - mean * mean, 0.0)
    rstd = lax.rsqrt(var + 1e-5)
    scale = gamma.astype(f32)[None, :] * rstd
    shift = beta.astype(f32)[None, :] - mean * scale

    rs = 20 if L % 20 == 0 else L
    scale_bc = jnp.tile(scale, (1, rs)).reshape(B, rs * cin, 1)
    shift_bc = jnp.tile(shift, (1, rs)).reshape(B, rs * cin, 1)
    w_blk = jnp.kron(jnp.eye(rs, dtype=f32), w_conv.astype(f32))
    b_col = jnp.tile(b_conv.astype(f32), rs).reshape(rs * d, 1)

    x_out = pl.pallas_call(
        _norm_elu_conv_kernel,
        out_shape=jax.ShapeDtypeStruct((B, L, d, L), f32),
        grid=(B, L // rs),
        in_specs=[
            pl.BlockSpec((None, rs, cin, L), lambda b, t: (b, t, 0, 0)),
            pl.BlockSpec((None, rs * cin, 1), lambda b, t: (b, 0, 0)),
            pl.BlockSpec((None, rs * cin, 1), lambda b, t: (b, 0, 0)),
            pl.BlockSpec((rs * d, rs * cin), lambda b, t: (0, 0)),
            pl.BlockSpec((rs * d, 1), lambda b, t: (0, 0)),
        ],
        out_specs=pl.BlockSpec((None, rs, d, L), lambda b, t: (b, t, 0, 0)),
        compiler_params=pltpu.CompilerParams(
            dimension_semantics=("parallel", "parallel"),
            vmem_limit_bytes=64 * 1024 * 1024,
        ),
    )(x_t, scale_bc, shift_bc, w_blk, b_col)
    x = jnp.transpose(x_out, (0, 1, 3, 2))

    Bm, Nm, Lm = msa.shape
    demb = msa_emb.shape[-1]
    vocab = table.shape[0]
    emb_t = jnp.transpose(msa_emb.astype(f32), (0, 1, 3, 2))

    ns = 8 if Nm % 8 == 0 else Nm
    wle_blk = jnp.kron(jnp.eye(ns, dtype=f32), w_le.astype(f32))
    ble_col = jnp.tile(b_le.astype(f32), ns).reshape(ns * d, 1)
    eye_ns = jnp.eye(ns, dtype=f32)
    c_cat = jnp.concatenate(
        [jnp.kron(eye_ns, table[t].astype(f32)[:, None]) for t in range(vocab)],
        axis=1)

    import functools
    m_out = pl.pallas_call(
        functools.partial(_msa_embed_kernel, vocab=vocab),
        out_shape=jax.ShapeDtypeStruct((Bm, Nm, d, Lm), f32),
        grid=(Bm, Nm // ns),
        in_specs=[
            pl.BlockSpec((None, ns, Lm), lambda b, t: (b, t, 0)),
            pl.BlockSpec((None, ns, demb, Lm), lambda b, t: (b, t, 0, 0)),
            pl.BlockSpec((ns * d, ns * demb), lambda b, t: (0, 0)),
            pl.BlockSpec((ns * d, vocab * ns), lambda b, t: (0, 0)),
            pl.BlockSpec((ns * d, 1), lambda b, t: (0, 0)),
        ],
        out_specs=pl.BlockSpec((None, ns, d, Lm), lambda b, t: (b, t, 0, 0)),
        compiler_params=pltpu.CompilerParams(
            dimension_semantics=("parallel", "parallel"),
            vmem_limit_bytes=64 * 1024 * 1024,
        ),
    )(msa.astype(jnp.int32), emb_t, wle_blk, c_cat, ble_col)
    m = jnp.transpose(m_out, (0, 1, 3, 2))

    return x, m


# speedup vs baseline: 5.8448x; 2.1153x over previous
import functools

import jax
import jax.numpy as jnp
from jax import lax
from jax.experimental import pallas as pl
from jax.experimental.pallas import tpu as pltpu


def _norm_elu_conv_kernel(x_ref, g_ref, be_ref, w_ref, b_ref, o_ref, *, cs):
    Lr, cin, Lc = x_ref.shape
    rows = Lr * cin
    x = x_ref[...].reshape(rows, Lc)

    rmod = lax.broadcasted_iota(jnp.int32, (cin, rows), 1) % cin
    cidx = lax.broadcasted_iota(jnp.int32, (cin, rows), 0)
    sel = (rmod == cidx).astype(jnp.float32)
    s1 = jnp.dot(sel, x, preferred_element_type=jnp.float32)
    s2 = jnp.dot(sel, x * x, preferred_element_type=jnp.float32)
    inv_n = 1.0 / (Lr * Lc)
    mean = jnp.sum(s1, axis=1, keepdims=True) * inv_n
    ex2 = jnp.sum(s2, axis=1, keepdims=True) * inv_n
    var = jnp.maximum(ex2 - mean * mean, 0.0)
    rstd = lax.rsqrt(var + 1e-5)
    scale = g_ref[...] * rstd
    shift = be_ref[...] - mean * scale

    rmod_t = lax.broadcasted_iota(jnp.int32, (rows, cin), 0) % cin
    cidx_t = lax.broadcasted_iota(jnp.int32, (rows, cin), 1)
    sel_t = (rmod_t == cidx_t).astype(jnp.float32)
    scale_col = jnp.dot(sel_t, scale, preferred_element_type=jnp.float32)
    shift_col = jnp.dot(sel_t, shift, preferred_element_type=jnp.float32)

    xa = x * scale_col + shift_col
    xe = jnp.where(xa > 0, xa, jnp.exp(jnp.minimum(xa, 0.0)) - 1.0)

    d = o_ref.shape[1]
    crows = cs * cin
    for i in range(Lr // cs):
        y = jnp.dot(w_ref[...], xe[i * crows:(i + 1) * crows, :],
                    preferred_element_type=jnp.float32) + b_ref[...]
        o_ref[i * cs:(i + 1) * cs] = y.reshape(cs, d, Lc)


def _msa_embed_kernel(tok_ref, emb_ref, w_ref, c_ref, b_ref, o_ref,
                      *, vocab, cs):
    ns, demb, L = emb_ref.shape
    d = o_ref.shape[1]
    e = emb_ref[...].reshape(ns * demb, L)
    tok = tok_ref[...]
    for i in range(ns // cs):
        masks = jnp.concatenate(
            [(tok[i * cs:(i + 1) * cs] == t).astype(jnp.float32)
             for t in range(vocab)], axis=0)
        y = (jnp.dot(w_ref[...], e[i * cs * demb:(i + 1) * cs * demb, :],
                     preferred_element_type=jnp.float32)
             + jnp.dot(c_ref[...], masks, preferred_element_type=jnp.float32)
             + b_ref[...])
        o_ref[i * cs:(i + 1) * cs] = y.reshape(cs, d, L)


def kernel(f2d, msa, msa_emb, gamma, beta, w_conv, b_conv, table, w_le, b_le):
    f32 = jnp.float32
    B, L, _, cin = f2d.shape
    d = w_conv.shape[0]

    x_t = jnp.transpose(f2d.astype(f32), (0, 1, 3, 2))

    cs = 20 if L % 20 == 0 else L
    w_blk = jnp.kron(jnp.eye(cs, dtype=f32), w_conv.astype(f32))
    b_col = jnp.tile(b_conv.astype(f32), cs).reshape(cs * d, 1)

    x_out = pl.pallas_call(
        functools.partial(_norm_elu_conv_kernel, cs=cs),
        out_shape=jax.ShapeDtypeStruct((B, L, d, L), f32),
        grid=(B,),
        in_specs=[
            pl.BlockSpec((None, L, cin, L), lambda b: (b, 0, 0, 0)),
            pl.BlockSpec((cin, 1), lambda b: (0, 0)),
            pl.BlockSpec((cin, 1), lambda b: (0, 0)),
            pl.BlockSpec((cs * d, cs * cin), lambda b: (0, 0)),
            pl.BlockSpec((cs * d, 1), lambda b: (0, 0)),
        ],
        out_specs=pl.BlockSpec((None, L, d, L), lambda b: (b, 0, 0, 0)),
        compiler_params=pltpu.CompilerParams(
            dimension_semantics=("parallel",),
            vmem_limit_bytes=100 * 1024 * 1024,
        ),
    )(x_t, gamma.astype(f32).reshape(cin, 1), beta.astype(f32).reshape(cin, 1),
      w_blk, b_col)
    x = jnp.transpose(x_out, (0, 1, 3, 2))

    Bm, Nm, Lm = msa.shape
    demb = msa_emb.shape[-1]
    vocab = table.shape[0]
    emb_t = jnp.transpose(msa_emb.astype(f32), (0, 1, 3, 2))

    ns = 16 if Nm % 16 == 0 else Nm
    csn = 8 if ns % 8 == 0 else ns
    wle_blk = jnp.kron(jnp.eye(csn, dtype=f32), w_le.astype(f32))
    ble_col = jnp.tile(b_le.astype(f32), csn).reshape(csn * d, 1)
    eye_cs = jnp.eye(csn, dtype=f32)
    c_cat = jnp.concatenate(
        [jnp.kron(eye_cs, table[t].astype(f32)[:, None]) for t in range(vocab)],
        axis=1)

    m_out = pl.pallas_call(
        functools.partial(_msa_embed_kernel, vocab=vocab, cs=csn),
        out_shape=jax.ShapeDtypeStruct((Bm, Nm, d, Lm), f32),
        grid=(Bm, Nm // ns),
        in_specs=[
            pl.BlockSpec((None, ns, Lm), lambda b, t: (b, t, 0)),
            pl.BlockSpec((None, ns, demb, Lm), lambda b, t: (b, t, 0, 0)),
            pl.BlockSpec((csn * d, csn * demb), lambda b, t: (0, 0)),
            pl.BlockSpec((csn * d, vocab * csn), lambda b, t: (0, 0)),
            pl.BlockSpec((csn * d, 1), lambda b, t: (0, 0)),
        ],
        out_specs=pl.BlockSpec((None, ns, d, Lm), lambda b, t: (b, t, 0, 0)),
        compiler_params=pltpu.CompilerParams(
            dimension_semantics=("parallel", "parallel"),
            vmem_limit_bytes=64 * 1024 * 1024,
        ),
    )(msa.astype(jnp.int32), emb_t, wle_blk, c_cat, ble_col)
    m = jnp.transpose(m_out, (0, 1, 3, 2))

    return x, m


# speedup vs baseline: 7.5486x; 1.2915x over previous
import functools

import jax
import jax.numpy as jnp
from jax import lax
from jax.experimental import pallas as pl
from jax.experimental.pallas import tpu as pltpu


def _fused_kernel(x_ref, g_ref, be_ref, w_ref, b_ref,
                  tok_ref, emb_ref, wle_ref, c_ref, ble_ref,
                  xo_ref, mo_ref, *, cs, csn, vocab):
    Lr, cin, Lc = x_ref.shape
    rows = Lr * cin
    x = x_ref[...].reshape(rows, Lc)

    rmod = lax.broadcasted_iota(jnp.int32, (cin, rows), 1) % cin
    cidx = lax.broadcasted_iota(jnp.int32, (cin, rows), 0)
    sel = (rmod == cidx).astype(jnp.float32)
    s1 = jnp.dot(sel, x, preferred_element_type=jnp.float32)
    s2 = jnp.dot(sel, x * x, preferred_element_type=jnp.float32)
    inv_n = 1.0 / (Lr * Lc)
    mean = jnp.sum(s1, axis=1, keepdims=True) * inv_n
    ex2 = jnp.sum(s2, axis=1, keepdims=True) * inv_n
    var = jnp.maximum(ex2 - mean * mean, 0.0)
    rstd = lax.rsqrt(var + 1e-5)
    scale = g_ref[...] * rstd
    shift = be_ref[...] - mean * scale

    rmod_t = lax.broadcasted_iota(jnp.int32, (rows, cin), 0) % cin
    cidx_t = lax.broadcasted_iota(jnp.int32, (rows, cin), 1)
    sel_t = (rmod_t == cidx_t).astype(jnp.float32)
    scale_col = jnp.dot(sel_t, scale, preferred_element_type=jnp.float32)
    shift_col = jnp.dot(sel_t, shift, preferred_element_type=jnp.float32)

    xa = x * scale_col + shift_col
    xe = jnp.where(xa > 0, xa, jnp.exp(jnp.minimum(xa, 0.0)) - 1.0)

    d = xo_ref.shape[1]
    crows = cs * cin
    for i in range(Lr // cs):
        y = jnp.dot(w_ref[...], xe[i * crows:(i + 1) * crows, :],
                    preferred_element_type=jnp.float32) + b_ref[...]
        xo_ref[i * cs:(i + 1) * cs] = y.reshape(cs, d, Lc)

    N, demb, Lm = emb_ref.shape
    e = emb_ref[...].reshape(N * demb, Lm)
    tok = tok_ref[...]
    for i in range(N // csn):
        masks = jnp.concatenate(
            [(tok[i * csn:(i + 1) * csn] == t).astype(jnp.float32)
             for t in range(vocab)], axis=0)
        y = (jnp.dot(wle_ref[...], e[i * csn * demb:(i + 1) * csn * demb, :],
                     preferred_element_type=jnp.float32)
             + jnp.dot(c_ref[...], masks, preferred_element_type=jnp.float32)
             + ble_ref[...])
        mo_ref[i * csn:(i + 1) * csn] = y.reshape(csn, d, Lm)


def kernel(f2d, msa, msa_emb, gamma, beta, w_conv, b_conv, table, w_le, b_le):
    f32 = jnp.float32
    B, L, _, cin = f2d.shape
    d = w_conv.shape[0]
    Bm, Nm, Lm = msa.shape
    demb = msa_emb.shape[-1]
    vocab = table.shape[0]

    x_t = jnp.transpose(f2d.astype(f32), (0, 1, 3, 2))
    emb_t = jnp.transpose(msa_emb.astype(f32), (0, 1, 3, 2))

    cs = 20 if L % 20 == 0 else L
    w_blk = jnp.kron(jnp.eye(cs, dtype=f32), w_conv.astype(f32))
    b_col = jnp.tile(b_conv.astype(f32), cs).reshape(cs * d, 1)

    csn = 8 if Nm % 8 == 0 else Nm
    wle_blk = jnp.kron(jnp.eye(csn, dtype=f32), w_le.astype(f32))
    ble_col = jnp.tile(b_le.astype(f32), csn).reshape(csn * d, 1)
    eye_cs = jnp.eye(csn, dtype=f32)
    c_cat = jnp.concatenate(
        [jnp.kron(eye_cs, table[t].astype(f32)[:, None]) for t in range(vocab)],
        axis=1)

    x_out, m_out = pl.pallas_call(
        functools.partial(_fused_kernel, cs=cs, csn=csn, vocab=vocab),
        out_shape=(jax.ShapeDtypeStruct((B, L, d, L), f32),
                   jax.ShapeDtypeStruct((Bm, Nm, d, Lm), f32)),
        grid=(B,),
        in_specs=[
            pl.BlockSpec((None, L, cin, L), lambda b: (b, 0, 0, 0)),
            pl.BlockSpec((cin, 1), lambda b: (0, 0)),
            pl.BlockSpec((cin, 1), lambda b: (0, 0)),
            pl.BlockSpec((cs * d, cs * cin), lambda b: (0, 0)),
            pl.BlockSpec((cs * d, 1), lambda b: (0, 0)),
            pl.BlockSpec((None, Nm, Lm), lambda b: (b, 0, 0)),
            pl.BlockSpec((None, Nm, demb, Lm), lambda b: (b, 0, 0, 0)),
            pl.BlockSpec((csn * d, csn * demb), lambda b: (0, 0)),
            pl.BlockSpec((csn * d, vocab * csn), lambda b: (0, 0)),
            pl.BlockSpec((csn * d, 1), lambda b: (0, 0)),
        ],
        out_specs=(pl.BlockSpec((None, L, d, L), lambda b: (b, 0, 0, 0)),
                   pl.BlockSpec((None, Nm, d, Lm), lambda b: (b, 0, 0, 0))),
        compiler_params=pltpu.CompilerParams(
            dimension_semantics=("parallel",),
            vmem_limit_bytes=100 * 1024 * 1024,
        ),
    )(x_t, gamma.astype(f32).reshape(cin, 1), beta.astype(f32).reshape(cin, 1),
      w_blk, b_col, msa.astype(jnp.int32), emb_t, wle_blk, c_cat, ble_col)

    x = jnp.transpose(x_out, (0, 1, 3, 2))
    m = jnp.transpose(m_out, (0, 1, 3, 2))
    return x, m


# speedup vs baseline: 7.7605x; 1.0281x over previous
import functools

import jax
import jax.numpy as jnp
from jax import lax
from jax.experimental import pallas as pl
from jax.experimental.pallas import tpu as pltpu


def _fused_kernel(x_ref, g_ref, be_ref, w_ref, b_ref,
                  tok_ref, emb_ref, wle_ref, c_ref, ble_ref,
                  xo_ref, mo_ref, *, cs, csn, vocab):
    Lr, cin, Lc = x_ref.shape
    rows = Lr * cin
    x = x_ref[...].reshape(rows, Lc)

    rmod = lax.broadcasted_iota(jnp.int32, (cin, rows), 1) % cin
    cidx = lax.broadcasted_iota(jnp.int32, (cin, rows), 0)
    sel = (rmod == cidx).astype(jnp.float32)
    s1 = jnp.dot(sel, x, preferred_element_type=jnp.float32)
    s2 = jnp.dot(sel, x * x, preferred_element_type=jnp.float32)
    inv_n = 1.0 / (Lr * Lc)
    mean = jnp.sum(s1, axis=1, keepdims=True) * inv_n
    ex2 = jnp.sum(s2, axis=1, keepdims=True) * inv_n
    var = jnp.maximum(ex2 - mean * mean, 0.0)
    rstd = lax.rsqrt(var + 1e-5)
    scale = g_ref[...] * rstd
    shift = be_ref[...] - mean * scale

    rmod_t = lax.broadcasted_iota(jnp.int32, (rows, cin), 0) % cin
    cidx_t = lax.broadcasted_iota(jnp.int32, (rows, cin), 1)
    sel_t = (rmod_t == cidx_t).astype(jnp.float32)
    scale_col = jnp.dot(sel_t, scale, preferred_element_type=jnp.float32)
    shift_col = jnp.dot(sel_t, shift, preferred_element_type=jnp.float32)

    xa = x * scale_col + shift_col
    xe = jnp.where(xa > 0, xa, jnp.exp(jnp.minimum(xa, 0.0)) - 1.0)

    d = xo_ref.shape[1]
    crows = cs * cin
    for i in range(Lr // cs):
        y = jnp.dot(w_ref[...], xe[i * crows:(i + 1) * crows, :],
                    preferred_element_type=jnp.float32) + b_ref[...]
        xo_ref[i * cs:(i + 1) * cs] = y.reshape(cs, d, Lc)

    N, demb, Lm = emb_ref.shape
    e = emb_ref[...].reshape(N * demb, Lm)
    tok = tok_ref[...]
    for i in range(N // csn):
        masks = jnp.concatenate(
            [(tok[i * csn:(i + 1) * csn] == t).astype(jnp.float32)
             for t in range(vocab)], axis=0)
        y = (jnp.dot(wle_ref[...], e[i * csn * demb:(i + 1) * csn * demb, :],
                     preferred_element_type=jnp.float32)
             + jnp.dot(c_ref[...], masks, preferred_element_type=jnp.float32)
             + ble_ref[...])
        mo_ref[i * csn:(i + 1) * csn] = y.reshape(csn, d, Lm)


def kernel(f2d, msa, msa_emb, gamma, beta, w_conv, b_conv, table, w_le, b_le):
    f32 = jnp.float32
    B, L, _, cin = f2d.shape
    d = w_conv.shape[0]
    Bm, Nm, Lm = msa.shape
    demb = msa_emb.shape[-1]
    vocab = table.shape[0]

    x_t = jnp.transpose(f2d.astype(f32), (0, 1, 3, 2))
    emb_t = jnp.transpose(msa_emb.astype(f32), (0, 1, 3, 2))

    cs = 10 if L % 10 == 0 else L
    w_blk = jnp.kron(jnp.eye(cs, dtype=f32), w_conv.astype(f32))
    b_col = jnp.tile(b_conv.astype(f32), cs).reshape(cs * d, 1)

    csn = 8 if Nm % 8 == 0 else Nm
    wle_blk = jnp.kron(jnp.eye(csn, dtype=f32), w_le.astype(f32))
    ble_col = jnp.tile(b_le.astype(f32), csn).reshape(csn * d, 1)
    eye_cs = jnp.eye(csn, dtype=f32)
    c_cat = jnp.concatenate(
        [jnp.kron(eye_cs, table[t].astype(f32)[:, None]) for t in range(vocab)],
        axis=1)

    x_out, m_out = pl.pallas_call(
        functools.partial(_fused_kernel, cs=cs, csn=csn, vocab=vocab),
        out_shape=(jax.ShapeDtypeStruct((B, L, d, L), f32),
                   jax.ShapeDtypeStruct((Bm, Nm, d, Lm), f32)),
        grid=(B,),
        in_specs=[
            pl.BlockSpec((None, L, cin, L), lambda b: (b, 0, 0, 0)),
            pl.BlockSpec((cin, 1), lambda b: (0, 0)),
            pl.BlockSpec((cin, 1), lambda b: (0, 0)),
            pl.BlockSpec((cs * d, cs * cin), lambda b: (0, 0)),
            pl.BlockSpec((cs * d, 1), lambda b: (0, 0)),
            pl.BlockSpec((None, Nm, Lm), lambda b: (b, 0, 0)),
            pl.BlockSpec((None, Nm, demb, Lm), lambda b: (b, 0, 0, 0)),
            pl.BlockSpec((csn * d, csn * demb), lambda b: (0, 0)),
            pl.BlockSpec((csn * d, vocab * csn), lambda b: (0, 0)),
            pl.BlockSpec((csn * d, 1), lambda b: (0, 0)),
        ],
        out_specs=(pl.BlockSpec((None, L, d, L), lambda b: (b, 0, 0, 0)),
                   pl.BlockSpec((None, Nm, d, Lm), lambda b: (b, 0, 0, 0))),
        compiler_params=pltpu.CompilerParams(
            dimension_semantics=("parallel",),
            vmem_limit_bytes=100 * 1024 * 1024,
        ),
    )(x_t, gamma.astype(f32).reshape(cin, 1), beta.astype(f32).reshape(cin, 1),
      w_blk, b_col, msa.astype(jnp.int32), emb_t, wle_blk, c_cat, ble_col)

    x = jnp.transpose(x_out, (0, 1, 3, 2))
    m = jnp.transpose(m_out, (0, 1, 3, 2))
    return x, m


# speedup vs baseline: 7.8106x; 1.0065x over previous
import functools

import jax
import jax.numpy as jnp
from jax import lax
from jax.experimental import pallas as pl
from jax.experimental.pallas import tpu as pltpu


def _fused_kernel(x_ref, g_ref, be_ref, w_ref, b_ref,
                  tok_ref, emb_ref, wle_ref, c_ref, ble_ref,
                  xo_ref, mo_ref, *, cs, csn, vocab):
    Lr, cin, Lc = x_ref.shape
    rows = Lr * cin
    x = x_ref[...].reshape(rows, Lc)

    rmod = lax.broadcasted_iota(jnp.int32, (cin, rows), 1) % cin
    cidx = lax.broadcasted_iota(jnp.int32, (cin, rows), 0)
    sel = (rmod == cidx).astype(jnp.float32)
    s1 = jnp.dot(sel, x, preferred_element_type=jnp.float32)
    s2 = jnp.dot(sel, x * x, preferred_element_type=jnp.float32)
    inv_n = 1.0 / (Lr * Lc)
    mean = jnp.sum(s1, axis=1, keepdims=True) * inv_n
    ex2 = jnp.sum(s2, axis=1, keepdims=True) * inv_n
    var = jnp.maximum(ex2 - mean * mean, 0.0)
    rstd = lax.rsqrt(var + 1e-5)
    scale = g_ref[...] * rstd
    shift = be_ref[...] - mean * scale

    rmod_t = lax.broadcasted_iota(jnp.int32, (rows, cin), 0) % cin
    cidx_t = lax.broadcasted_iota(jnp.int32, (rows, cin), 1)
    sel_t = (rmod_t == cidx_t).astype(jnp.float32)
    scale_col = jnp.dot(sel_t, scale, preferred_element_type=jnp.float32)
    shift_col = jnp.dot(sel_t, shift, preferred_element_type=jnp.float32)

    xa = x * scale_col + shift_col
    xe = jnp.where(xa > 0, xa, jnp.exp(jnp.minimum(xa, 0.0)) - 1.0)

    d = xo_ref.shape[1]
    crows = cs * cin
    for i in range(Lr // cs):
        y = jnp.dot(w_ref[...], xe[i * crows:(i + 1) * crows, :],
                    preferred_element_type=jnp.float32) + b_ref[...]
        xo_ref[i * cs:(i + 1) * cs] = y.reshape(cs, d, Lc)

    N, demb, Lm = emb_ref.shape
    e = emb_ref[...].reshape(N * demb, Lm)
    tok = tok_ref[...]
    for i in range(N // csn):
        masks = jnp.concatenate(
            [(tok[i * csn:(i + 1) * csn] == t).astype(jnp.float32)
             for t in range(vocab)], axis=0)
        y = (jnp.dot(wle_ref[...], e[i * csn * demb:(i + 1) * csn * demb, :],
                     preferred_element_type=jnp.float32)
             + jnp.dot(c_ref[...], masks, preferred_element_type=jnp.float32)
             + ble_ref[...])
        mo_ref[i * csn:(i + 1) * csn] = y.reshape(csn, d, Lm)


def kernel(f2d, msa, msa_emb, gamma, beta, w_conv, b_conv, table, w_le, b_le):
    f32 = jnp.float32
    B, L, _, cin = f2d.shape
    d = w_conv.shape[0]
    Bm, Nm, Lm = msa.shape
    demb = msa_emb.shape[-1]
    vocab = table.shape[0]

    x_t = jnp.transpose(f2d.astype(f32), (0, 1, 3, 2))
    emb_t = jnp.transpose(msa_emb.astype(f32), (0, 1, 3, 2))

    cs = 10 if L % 10 == 0 else L
    w_blk = jnp.kron(jnp.eye(cs, dtype=f32), w_conv.astype(f32))
    b_col = jnp.tile(b_conv.astype(f32), cs).reshape(cs * d, 1)

    csn = 4 if Nm % 4 == 0 else Nm
    wle_blk = jnp.kron(jnp.eye(csn, dtype=f32), w_le.astype(f32))
    ble_col = jnp.tile(b_le.astype(f32), csn).reshape(csn * d, 1)
    eye_cs = jnp.eye(csn, dtype=f32)
    c_cat = jnp.concatenate(
        [jnp.kron(eye_cs, table[t].astype(f32)[:, None]) for t in range(vocab)],
        axis=1)

    x_out, m_out = pl.pallas_call(
        functools.partial(_fused_kernel, cs=cs, csn=csn, vocab=vocab),
        out_shape=(jax.ShapeDtypeStruct((B, L, d, L), f32),
                   jax.ShapeDtypeStruct((Bm, Nm, d, Lm), f32)),
        grid=(B,),
        in_specs=[
            pl.BlockSpec((None, L, cin, L), lambda b: (b, 0, 0, 0)),
            pl.BlockSpec((cin, 1), lambda b: (0, 0)),
            pl.BlockSpec((cin, 1), lambda b: (0, 0)),
            pl.BlockSpec((cs * d, cs * cin), lambda b: (0, 0)),
            pl.BlockSpec((cs * d, 1), lambda b: (0, 0)),
            pl.BlockSpec((None, Nm, Lm), lambda b: (b, 0, 0)),
            pl.BlockSpec((None, Nm, demb, Lm), lambda b: (b, 0, 0, 0)),
            pl.BlockSpec((csn * d, csn * demb), lambda b: (0, 0)),
            pl.BlockSpec((csn * d, vocab * csn), lambda b: (0, 0)),
            pl.BlockSpec((csn * d, 1), lambda b: (0, 0)),
        ],
        out_specs=(pl.BlockSpec((None, L, d, L), lambda b: (b, 0, 0, 0)),
                   pl.BlockSpec((None, Nm, d, Lm), lambda b: (b, 0, 0, 0))),
        compiler_params=pltpu.CompilerParams(
            dimension_semantics=("parallel",),
            vmem_limit_bytes=100 * 1024 * 1024,
        ),
    )(x_t, gamma.astype(f32).reshape(cin, 1), beta.astype(f32).reshape(cin, 1),
      w_blk, b_col, msa.astype(jnp.int32), emb_t, wle_blk, c_cat, ble_col)

    x = jnp.transpose(x_out, (0, 1, 3, 2))
    m = jnp.transpose(m_out, (0, 1, 3, 2))
    return x, m


# speedup vs baseline: 8.0367x; 1.0289x over previous
import functools

import jax
import jax.numpy as jnp
from jax import lax
from jax.experimental import pallas as pl
from jax.experimental.pallas import tpu as pltpu


def _fused_kernel(x_ref, g_ref, be_ref, w_ref, b_ref,
                  tok_ref, emb_ref, wle_ref, c_ref, ble_ref,
                  xo_ref, mo_ref, *, cs, csn, vocab):
    for e in range(x_ref.shape[0]):
        _one_element(x_ref, g_ref, be_ref, w_ref, b_ref, tok_ref, emb_ref,
                     wle_ref, c_ref, ble_ref, xo_ref, mo_ref, e,
                     cs=cs, csn=csn, vocab=vocab)


def _one_element(x_ref, g_ref, be_ref, w_ref, b_ref,
                 tok_ref, emb_ref, wle_ref, c_ref, ble_ref,
                 xo_ref, mo_ref, e, *, cs, csn, vocab):
    _, Lr, cin, Lc = x_ref.shape
    rows = Lr * cin
    x = x_ref[e].reshape(rows, Lc)

    rmod = lax.broadcasted_iota(jnp.int32, (cin, rows), 1) % cin
    cidx = lax.broadcasted_iota(jnp.int32, (cin, rows), 0)
    sel = (rmod == cidx).astype(jnp.float32)
    s1 = jnp.dot(sel, x, preferred_element_type=jnp.float32)
    s2 = jnp.dot(sel, x * x, preferred_element_type=jnp.float32)
    inv_n = 1.0 / (Lr * Lc)
    mean = jnp.sum(s1, axis=1, keepdims=True) * inv_n
    ex2 = jnp.sum(s2, axis=1, keepdims=True) * inv_n
    var = jnp.maximum(ex2 - mean * mean, 0.0)
    rstd = lax.rsqrt(var + 1e-5)
    scale = g_ref[...] * rstd
    shift = be_ref[...] - mean * scale

    rmod_t = lax.broadcasted_iota(jnp.int32, (rows, cin), 0) % cin
    cidx_t = lax.broadcasted_iota(jnp.int32, (rows, cin), 1)
    sel_t = (rmod_t == cidx_t).astype(jnp.float32)
    scale_col = jnp.dot(sel_t, scale, preferred_element_type=jnp.float32)
    shift_col = jnp.dot(sel_t, shift, preferred_element_type=jnp.float32)

    xa = x * scale_col + shift_col
    xe = jnp.where(xa > 0, xa, jnp.exp(jnp.minimum(xa, 0.0)) - 1.0)

    d = xo_ref.shape[2]
    crows = cs * cin
    for i in range(Lr // cs):
        y = jnp.dot(w_ref[...], xe[i * crows:(i + 1) * crows, :],
                    preferred_element_type=jnp.float32) + b_ref[...]
        xo_ref[e, i * cs:(i + 1) * cs] = y.reshape(cs, d, Lc)

    _, N, demb, Lm = emb_ref.shape
    emb = emb_ref[e].reshape(N * demb, Lm)
    tok = tok_ref[e]
    for i in range(N // csn):
        masks = jnp.concatenate(
            [(tok[i * csn:(i + 1) * csn] == t).astype(jnp.float32)
             for t in range(vocab)], axis=0)
        y = (jnp.dot(wle_ref[...], emb[i * csn * demb:(i + 1) * csn * demb, :],
                     preferred_element_type=jnp.float32)
             + jnp.dot(c_ref[...], masks, preferred_element_type=jnp.float32)
             + ble_ref[...])
        mo_ref[e, i * csn:(i + 1) * csn] = y.reshape(csn, d, Lm)


def kernel(f2d, msa, msa_emb, gamma, beta, w_conv, b_conv, table, w_le, b_le):
    f32 = jnp.float32
    B, L, _, cin = f2d.shape
    d = w_conv.shape[0]
    Bm, Nm, Lm = msa.shape
    demb = msa_emb.shape[-1]
    vocab = table.shape[0]

    x_t = jnp.transpose(f2d.astype(f32), (0, 1, 3, 2))
    emb_t = jnp.transpose(msa_emb.astype(f32), (0, 1, 3, 2))

    cs = 10 if L % 10 == 0 else L
    w_blk = jnp.kron(jnp.eye(cs, dtype=f32), w_conv.astype(f32))
    b_col = jnp.tile(b_conv.astype(f32), cs).reshape(cs * d, 1)

    csn = 4 if Nm % 4 == 0 else Nm
    wle_blk = jnp.kron(jnp.eye(csn, dtype=f32), w_le.astype(f32))
    ble_col = jnp.tile(b_le.astype(f32), csn).reshape(csn * d, 1)
    eye_cs = jnp.eye(csn, dtype=f32)
    c_cat = jnp.concatenate(
        [jnp.kron(eye_cs, table[t].astype(f32)[:, None]) for t in range(vocab)],
        axis=1)

    eb = 2 if B % 2 == 0 else 1
    x_out, m_out = pl.pallas_call(
        functools.partial(_fused_kernel, cs=cs, csn=csn, vocab=vocab),
        out_shape=(jax.ShapeDtypeStruct((B, L, d, L), f32),
                   jax.ShapeDtypeStruct((Bm, Nm, d, Lm), f32)),
        grid=(B // eb,),
        in_specs=[
            pl.BlockSpec((eb, L, cin, L), lambda b: (b, 0, 0, 0)),
            pl.BlockSpec((cin, 1), lambda b: (0, 0)),
            pl.BlockSpec((cin, 1), lambda b: (0, 0)),
            pl.BlockSpec((cs * d, cs * cin), lambda b: (0, 0)),
            pl.BlockSpec((cs * d, 1), lambda b: (0, 0)),
            pl.BlockSpec((eb, Nm, Lm), lambda b: (b, 0, 0)),
            pl.BlockSpec((eb, Nm, demb, Lm), lambda b: (b, 0, 0, 0)),
            pl.BlockSpec((csn * d, csn * demb), lambda b: (0, 0)),
            pl.BlockSpec((csn * d, vocab * csn), lambda b: (0, 0)),
            pl.BlockSpec((csn * d, 1), lambda b: (0, 0)),
        ],
        out_specs=(pl.BlockSpec((eb, L, d, L), lambda b: (b, 0, 0, 0)),
                   pl.BlockSpec((eb, Nm, d, Lm), lambda b: (b, 0, 0, 0))),
        compiler_params=pltpu.CompilerParams(
            dimension_semantics=("parallel",),
            vmem_limit_bytes=100 * 1024 * 1024,
        ),
    )(x_t, gamma.astype(f32).reshape(cin, 1), beta.astype(f32).reshape(cin, 1),
      w_blk, b_col, msa.astype(jnp.int32), emb_t, wle_blk, c_cat, ble_col)

    x = jnp.transpose(x_out, (0, 1, 3, 2))
    m = jnp.transpose(m_out, (0, 1, 3, 2))
    return x, m


# speedup vs baseline: 8.1405x; 1.0129x over previous
import functools

import jax
import jax.numpy as jnp
from jax import lax
from jax.experimental import pallas as pl
from jax.experimental.pallas import tpu as pltpu


def _fused_kernel(x_ref, g_ref, be_ref, w_ref, b_ref,
                  tok_ref, emb_ref, wle_ref, c_ref, ble_ref,
                  xo_ref, mo_ref, *, cs, csn, vocab):
    for e in range(x_ref.shape[0]):
        _one_element(x_ref, g_ref, be_ref, w_ref, b_ref, tok_ref, emb_ref,
                     wle_ref, c_ref, ble_ref, xo_ref, mo_ref, e,
                     cs=cs, csn=csn, vocab=vocab)


def _one_element(x_ref, g_ref, be_ref, w_ref, b_ref,
                 tok_ref, emb_ref, wle_ref, c_ref, ble_ref,
                 xo_ref, mo_ref, e, *, cs, csn, vocab):
    _, Lr, cin, Lc = x_ref.shape
    rows = Lr * cin
    x = x_ref[e].reshape(rows, Lc)

    rmod = lax.broadcasted_iota(jnp.int32, (cin, rows), 1) % cin
    cidx = lax.broadcasted_iota(jnp.int32, (cin, rows), 0)
    sel = (rmod == cidx).astype(jnp.float32)
    s1 = jnp.dot(sel, x, preferred_element_type=jnp.float32)
    s2 = jnp.dot(sel, x * x, preferred_element_type=jnp.float32)
    inv_n = 1.0 / (Lr * Lc)
    mean = jnp.sum(s1, axis=1, keepdims=True) * inv_n
    ex2 = jnp.sum(s2, axis=1, keepdims=True) * inv_n
    var = jnp.maximum(ex2 - mean * mean, 0.0)
    rstd = lax.rsqrt(var + 1e-5)
    scale = g_ref[...] * rstd
    shift = be_ref[...] - mean * scale

    rmod_t = lax.broadcasted_iota(jnp.int32, (rows, cin), 0) % cin
    cidx_t = lax.broadcasted_iota(jnp.int32, (rows, cin), 1)
    sel_t = (rmod_t == cidx_t).astype(jnp.float32)
    scale_col = jnp.dot(sel_t, scale, preferred_element_type=jnp.float32)
    shift_col = jnp.dot(sel_t, shift, preferred_element_type=jnp.float32)

    xa = x * scale_col + shift_col
    xe = jnp.where(xa > 0, xa, jnp.exp(jnp.minimum(xa, 0.0)) - 1.0)

    d = xo_ref.shape[2]
    crows = cs * cin
    for i in range(Lr // cs):
        y = jnp.dot(w_ref[...], xe[i * crows:(i + 1) * crows, :],
                    preferred_element_type=jnp.float32) + b_ref[...]
        xo_ref[e, i * cs:(i + 1) * cs] = y.reshape(cs, d, Lc)

    _, N, demb, Lm = emb_ref.shape
    emb = emb_ref[e].reshape(N * demb, Lm)
    tok = tok_ref[e]
    for i in range(N // csn):
        masks = jnp.concatenate(
            [(tok[i * csn:(i + 1) * csn] == t).astype(jnp.float32)
             for t in range(vocab)], axis=0)
        y = (jnp.dot(wle_ref[...], emb[i * csn * demb:(i + 1) * csn * demb, :],
                     preferred_element_type=jnp.float32)
             + jnp.dot(c_ref[...], masks, preferred_element_type=jnp.float32)
             + ble_ref[...])
        mo_ref[e, i * csn:(i + 1) * csn] = y.reshape(csn, d, Lm)


def kernel(f2d, msa, msa_emb, gamma, beta, w_conv, b_conv, table, w_le, b_le):
    f32 = jnp.float32
    B, L, _, cin = f2d.shape
    d = w_conv.shape[0]
    Bm, Nm, Lm = msa.shape
    demb = msa_emb.shape[-1]
    vocab = table.shape[0]

    x_t = jnp.transpose(f2d.astype(f32), (0, 1, 3, 2))
    emb_t = jnp.transpose(msa_emb.astype(f32), (0, 1, 3, 2))

    cs = 5 if L % 5 == 0 else L
    w_blk = jnp.kron(jnp.eye(cs, dtype=f32), w_conv.astype(f32))
    b_col = jnp.tile(b_conv.astype(f32), cs).reshape(cs * d, 1)

    csn = 4 if Nm % 4 == 0 else Nm
    wle_blk = jnp.kron(jnp.eye(csn, dtype=f32), w_le.astype(f32))
    ble_col = jnp.tile(b_le.astype(f32), csn).reshape(csn * d, 1)
    eye_cs = jnp.eye(csn, dtype=f32)
    c_cat = jnp.concatenate(
        [jnp.kron(eye_cs, table[t].astype(f32)[:, None]) for t in range(vocab)],
        axis=1)

    eb = 2 if B % 2 == 0 else 1
    x_out, m_out = pl.pallas_call(
        functools.partial(_fused_kernel, cs=cs, csn=csn, vocab=vocab),
        out_shape=(jax.ShapeDtypeStruct((B, L, d, L), f32),
                   jax.ShapeDtypeStruct((Bm, Nm, d, Lm), f32)),
        grid=(B // eb,),
        in_specs=[
            pl.BlockSpec((eb, L, cin, L), lambda b: (b, 0, 0, 0)),
            pl.BlockSpec((cin, 1), lambda b: (0, 0)),
            pl.BlockSpec((cin, 1), lambda b: (0, 0)),
            pl.BlockSpec((cs * d, cs * cin), lambda b: (0, 0)),
            pl.BlockSpec((cs * d, 1), lambda b: (0, 0)),
            pl.BlockSpec((eb, Nm, Lm), lambda b: (b, 0, 0)),
            pl.BlockSpec((eb, Nm, demb, Lm), lambda b: (b, 0, 0, 0)),
            pl.BlockSpec((csn * d, csn * demb), lambda b: (0, 0)),
            pl.BlockSpec((csn * d, vocab * csn), lambda b: (0, 0)),
            pl.BlockSpec((csn * d, 1), lambda b: (0, 0)),
        ],
        out_specs=(pl.BlockSpec((eb, L, d, L), lambda b: (b, 0, 0, 0)),
                   pl.BlockSpec((eb, Nm, d, Lm), lambda b: (b, 0, 0, 0))),
        compiler_params=pltpu.CompilerParams(
            dimension_semantics=("parallel",),
            vmem_limit_bytes=100 * 1024 * 1024,
        ),
    )(x_t, gamma.astype(f32).reshape(cin, 1), beta.astype(f32).reshape(cin, 1),
      w_blk, b_col, msa.astype(jnp.int32), emb_t, wle_blk, c_cat, ble_col)

    x = jnp.transpose(x_out, (0, 1, 3, 2))
    m = jnp.transpose(m_out, (0, 1, 3, 2))
    return x, m


# speedup vs baseline: 8.1797x; 1.0048x over previous
import functools

import jax
import jax.numpy as jnp
from jax import lax
from jax.experimental import pallas as pl
from jax.experimental.pallas import tpu as pltpu


def _fused_kernel(x_ref, g_ref, be_ref, w_ref, b_ref,
                  tok_ref, emb_ref, wle_ref, c_ref, ble_ref,
                  xo_ref, mo_ref, *, cs, csn, vocab):
    for e in range(x_ref.shape[0]):
        _one_element(x_ref, g_ref, be_ref, w_ref, b_ref, tok_ref, emb_ref,
                     wle_ref, c_ref, ble_ref, xo_ref, mo_ref, e,
                     cs=cs, csn=csn, vocab=vocab)


def _one_element(x_ref, g_ref, be_ref, w_ref, b_ref,
                 tok_ref, emb_ref, wle_ref, c_ref, ble_ref,
                 xo_ref, mo_ref, e, *, cs, csn, vocab):
    _, Lr, cin, Lc = x_ref.shape
    rows = Lr * cin
    x = x_ref[e].reshape(rows, Lc)

    rmod = lax.broadcasted_iota(jnp.int32, (cin, rows), 1) % cin
    cidx = lax.broadcasted_iota(jnp.int32, (cin, rows), 0)
    sel = (rmod == cidx).astype(jnp.float32)
    s1 = jnp.dot(sel, x, preferred_element_type=jnp.float32)
    s2 = jnp.dot(sel, x * x, preferred_element_type=jnp.float32)
    inv_n = 1.0 / (Lr * Lc)
    mean = jnp.sum(s1, axis=1, keepdims=True) * inv_n
    ex2 = jnp.sum(s2, axis=1, keepdims=True) * inv_n
    var = jnp.maximum(ex2 - mean * mean, 0.0)
    rstd = lax.rsqrt(var + 1e-5)
    scale = g_ref[...] * rstd
    shift = be_ref[...] - mean * scale

    rmod_t = lax.broadcasted_iota(jnp.int32, (rows, cin), 0) % cin
    cidx_t = lax.broadcasted_iota(jnp.int32, (rows, cin), 1)
    sel_t = (rmod_t == cidx_t).astype(jnp.float32)
    scale_col = jnp.dot(sel_t, scale, preferred_element_type=jnp.float32)
    shift_col = jnp.dot(sel_t, shift, preferred_element_type=jnp.float32)

    xa = x * scale_col + shift_col
    xe = jnp.where(xa > 0, xa, jnp.exp(jnp.minimum(xa, 0.0)) - 1.0)

    d = xo_ref.shape[2]
    crows = cs * cin
    for i in range(Lr // cs):
        y = jnp.dot(w_ref[...], xe[i * crows:(i + 1) * crows, :],
                    preferred_element_type=jnp.float32) + b_ref[...]
        xo_ref[e, i * cs:(i + 1) * cs] = y.reshape(cs, d, Lc)

    _, N, demb, Lm = emb_ref.shape
    emb = emb_ref[e].reshape(N * demb, Lm)
    tok = tok_ref[e]
    for i in range(N // csn):
        masks = jnp.concatenate(
            [(tok[i * csn:(i + 1) * csn] == t).astype(jnp.float32)
             for t in range(vocab)], axis=0)
        y = (jnp.dot(wle_ref[...], emb[i * csn * demb:(i + 1) * csn * demb, :],
                     preferred_element_type=jnp.float32)
             + jnp.dot(c_ref[...], masks, preferred_element_type=jnp.float32)
             + ble_ref[...])
        mo_ref[e, i * csn:(i + 1) * csn] = y.reshape(csn, d, Lm)


def kernel(f2d, msa, msa_emb, gamma, beta, w_conv, b_conv, table, w_le, b_le):
    f32 = jnp.float32
    B, L, _, cin = f2d.shape
    d = w_conv.shape[0]
    Bm, Nm, Lm = msa.shape
    demb = msa_emb.shape[-1]
    vocab = table.shape[0]

    x_t = jnp.transpose(f2d.astype(f32), (0, 1, 3, 2))
    emb_t = jnp.transpose(msa_emb.astype(f32), (0, 1, 3, 2))

    cs = 4 if L % 4 == 0 else L
    w_blk = jnp.kron(jnp.eye(cs, dtype=f32), w_conv.astype(f32))
    b_col = jnp.tile(b_conv.astype(f32), cs).reshape(cs * d, 1)

    csn = 4 if Nm % 4 == 0 else Nm
    wle_blk = jnp.kron(jnp.eye(csn, dtype=f32), w_le.astype(f32))
    ble_col = jnp.tile(b_le.astype(f32), csn).reshape(csn * d, 1)
    eye_cs = jnp.eye(csn, dtype=f32)
    c_cat = jnp.concatenate(
        [jnp.kron(eye_cs, table[t].astype(f32)[:, None]) for t in range(vocab)],
        axis=1)

    eb = 2 if B % 2 == 0 else 1
    x_out, m_out = pl.pallas_call(
        functools.partial(_fused_kernel, cs=cs, csn=csn, vocab=vocab),
        out_shape=(jax.ShapeDtypeStruct((B, L, d, L), f32),
                   jax.ShapeDtypeStruct((Bm, Nm, d, Lm), f32)),
        grid=(B // eb,),
        in_specs=[
            pl.BlockSpec((eb, L, cin, L), lambda b: (b, 0, 0, 0)),
            pl.BlockSpec((cin, 1), lambda b: (0, 0)),
            pl.BlockSpec((cin, 1), lambda b: (0, 0)),
            pl.BlockSpec((cs * d, cs * cin), lambda b: (0, 0)),
            pl.BlockSpec((cs * d, 1), lambda b: (0, 0)),
            pl.BlockSpec((eb, Nm, Lm), lambda b: (b, 0, 0)),
            pl.BlockSpec((eb, Nm, demb, Lm), lambda b: (b, 0, 0, 0)),
            pl.BlockSpec((csn * d, csn * demb), lambda b: (0, 0)),
            pl.BlockSpec((csn * d, vocab * csn), lambda b: (0, 0)),
            pl.BlockSpec((csn * d, 1), lambda b: (0, 0)),
        ],
        out_specs=(pl.BlockSpec((eb, L, d, L), lambda b: (b, 0, 0, 0)),
                   pl.BlockSpec((eb, Nm, d, Lm), lambda b: (b, 0, 0, 0))),
        compiler_params=pltpu.CompilerParams(
            dimension_semantics=("parallel",),
            vmem_limit_bytes=100 * 1024 * 1024,
        ),
    )(x_t, gamma.astype(f32).reshape(cin, 1), beta.astype(f32).reshape(cin, 1),
      w_blk, b_col, msa.astype(jnp.int32), emb_t, wle_blk, c_cat, ble_col)

    x = jnp.transpose(x_out, (0, 1, 3, 2))
    m = jnp.transpose(m_out, (0, 1, 3, 2))
    return x, m
